# Initial kernel scaffold; baseline (speedup 1.0000x reference)
import jax
import jax.numpy as jnp
from jax.experimental import pallas as pl


def kernel(x, edge_index, W1, b1, W2, b2, Wout, bout):
    raise NotImplementedError("write your pallas kernel here")


# speedup vs baseline: 11.4774x; 11.4774x over previous
import jax
import jax.numpy as jnp
from jax import lax
from jax.experimental import pallas as pl
from jax.experimental.pallas import tpu as pltpu
from jax.experimental.pallas import tpu_sc as plsc

_NC = 2
_NS = 16
_C = 128


def _ceil_to(v, m):
    return (v + m - 1) // m * m


def _mesh():
    return plsc.VectorSubcoreMesh(
        core_axis_name="c", subcore_axis_name="s",
        num_cores=_NC, num_subcores=_NS)


def _sc_degree(n_tab, e_pad):
    chunks = e_pad // (_NC * _NS * _C)
    rows = n_tab // _NS

    def body(dst_hbm, deg_out, dstv, onev, stagev, degacc):
        c = lax.axis_index("c")
        s = lax.axis_index("s")
        r0 = s * rows
        for i in range(rows // 16):
            stagev[pl.ds(i * 16, 16)] = jnp.zeros((16,), jnp.float32)
        for i in range(_C // 16):
            onev[pl.ds(i * 16, 16)] = jnp.ones((16,), jnp.float32)
        pltpu.sync_copy(stagev, degacc.at[pl.ds(r0, rows)])
        plsc.subcore_barrier()
        w = c * _NS + s

        def chunk(j, carry):
            base = (w * chunks + j) * _C
            pltpu.sync_copy(dst_hbm.at[pl.ds(base, _C)], dstv)
            pltpu.sync_copy(onev, degacc.at[dstv], add=True)
            return carry

        lax.fori_loop(0, chunks, chunk, 0)
        plsc.subcore_barrier()
        pltpu.sync_copy(degacc.at[pl.ds(r0, rows)], stagev)
        pltpu.sync_copy(stagev, deg_out.at[pl.ds(c * n_tab + r0, rows)])

    return pl.kernel(
        body,
        out_type=jax.ShapeDtypeStruct((_NC * n_tab,), jnp.float32),
        mesh=_mesh(),
        scratch_types=[
            pltpu.VMEM((_C,), jnp.int32),
            pltpu.VMEM((_C,), jnp.float32),
            pltpu.VMEM((rows,), jnp.float32),
            pltpu.VMEM_SHARED((n_tab,), jnp.float32),
        ],
    )


def _sc_aggregate(h, n_tab, e_pad):
    chunks = e_pad // (_NC * _NS * _C)
    rows = n_tab // _NS

    def body(hs_hbm, src_hbm, dst_hbm, acc_out, srcv, dstv, rowsv, acc, sem):
        c = lax.axis_index("c")
        s = lax.axis_index("s")
        r0 = s * rows

        def zrow(i, carry):
            for k in range(h // 16):
                rowsv[i, pl.ds(k * 16, 16)] = jnp.zeros((16,), jnp.float32)
            return carry

        lax.fori_loop(0, _C, zrow, 0)

        def zcp(i, carry):
            pltpu.sync_copy(rowsv, acc.at[pl.ds(r0 + i * _C, _C)])
            return carry

        lax.fori_loop(0, rows // _C, zcp, 0)
        plsc.subcore_barrier()
        w = c * _NS + s

        def chunk(j, carry):
            base = (w * chunks + j) * _C
            pltpu.sync_copy(src_hbm.at[pl.ds(base, _C)], srcv)
            pltpu.sync_copy(dst_hbm.at[pl.ds(base, _C)], dstv)
            pltpu.async_copy(hs_hbm.at[srcv], rowsv, sem).wait()
            pltpu.sync_copy(rowsv, acc.at[dstv], add=True)
            return carry

        lax.fori_loop(0, chunks, chunk, 0)
        plsc.subcore_barrier()

        def wb(i, carry):
            pltpu.sync_copy(acc.at[pl.ds(r0 + i * _C, _C)], rowsv)
            pltpu.sync_copy(rowsv, acc_out.at[pl.ds(c * n_tab + r0 + i * _C, _C)])
            return carry

        lax.fori_loop(0, rows // _C, wb, 0)

    return pl.kernel(
        body,
        out_type=jax.ShapeDtypeStruct((_NC * n_tab, h), jnp.float32),
        mesh=_mesh(),
        scratch_types=[
            pltpu.VMEM((_C,), jnp.int32),
            pltpu.VMEM((_C,), jnp.int32),
            pltpu.VMEM((_C, h), jnp.float32),
            pltpu.VMEM_SHARED((n_tab, h), jnp.float32),
            pltpu.SemaphoreType.DMA,
        ],
    )


def _tc_mm1(degp, x, w1, n_tab):
    n, _ = x.shape
    h = w1.shape[1]

    def body(degp_ref, x_ref, w1_ref, dinv_ref, hs_ref):
        deg = degp_ref[0, :] + degp_ref[1, :] + 1.0
        dinv = lax.rsqrt(deg)
        dinv_ref[...] = dinv[:, None]
        xw = jnp.dot(x_ref[...], w1_ref[...], preferred_element_type=jnp.float32)
        hs_ref[...] = xw * dinv[:n, None]

    return pl.pallas_call(
        body,
        out_shape=(jax.ShapeDtypeStruct((n_tab, 1), jnp.float32),
                   jax.ShapeDtypeStruct((n, h), jnp.float32)),
    )(degp, x, w1)


def _tc_layer(accp, hs, dinv, b, w2):
    n, h = hs.shape
    r = 2000 if n % 2000 == 0 else n
    grid = (n // r,)

    def body(accp_ref, hs_ref, dinv_ref, b_ref, w2_ref, out_ref):
        agg = accp_ref[0] + accp_ref[1] + hs_ref[...]
        dv = dinv_ref[...]
        h1 = jnp.maximum(agg * dv + b_ref[...], 0.0)
        out_ref[...] = jnp.dot(h1, w2_ref[...], preferred_element_type=jnp.float32) * dv

    return pl.pallas_call(
        body,
        grid=grid,
        in_specs=[
            pl.BlockSpec((2, r, h), lambda i: (0, i, 0)),
            pl.BlockSpec((r, h), lambda i: (i, 0)),
            pl.BlockSpec((r, 1), lambda i: (i, 0)),
            pl.BlockSpec((1, h), lambda i: (0, 0)),
            pl.BlockSpec((h, h), lambda i: (0, 0)),
        ],
        out_specs=pl.BlockSpec((r, h), lambda i: (i, 0)),
        out_shape=jax.ShapeDtypeStruct((n, h), jnp.float32),
    )(accp, hs, dinv, b, w2)


def _tc_head(accp, hs, dinv, b, wout, bout):
    n, h = hs.shape
    a = wout.shape[0]
    r = 2000 if n % 2000 == 0 else n
    grid = (n // r,)

    def body(accp_ref, hs_ref, dinv_ref, b_ref, wout_ref, bout_ref, q_ref):
        agg = accp_ref[0] + accp_ref[1] + hs_ref[...]
        dv = dinv_ref[...]
        h2 = jnp.maximum(agg * dv + b_ref[...], 0.0)
        q_ref[...] = lax.dot_general(
            h2, wout_ref[...], (((1,), (1,)), ((), ())),
            preferred_element_type=jnp.float32) + bout_ref[...]

    return pl.pallas_call(
        body,
        grid=grid,
        in_specs=[
            pl.BlockSpec((2, r, h), lambda i: (0, i, 0)),
            pl.BlockSpec((r, h), lambda i: (i, 0)),
            pl.BlockSpec((r, 1), lambda i: (i, 0)),
            pl.BlockSpec((1, h), lambda i: (0, 0)),
            pl.BlockSpec((a, h), lambda i: (0, 0)),
            pl.BlockSpec((1, a), lambda i: (0, 0)),
        ],
        out_specs=pl.BlockSpec((r, a), lambda i: (i, 0)),
        out_shape=jax.ShapeDtypeStruct((n, a), jnp.float32),
    )(accp, hs, dinv, b, wout, bout)


def kernel(x, edge_index, W1, b1, W2, b2, Wout, bout):
    n, _ = x.shape
    e = edge_index.shape[1]
    h = W1.shape[1]

    n_tab = _ceil_to(n, _NS * 16)
    e_pad = _ceil_to(e, _NC * _NS * _C)

    src = edge_index[0].astype(jnp.int32)
    dst = edge_index[1].astype(jnp.int32)
    pad = e_pad - e
    src_p = jnp.concatenate([src, jnp.zeros((pad,), jnp.int32)])
    dst_p = jnp.concatenate([dst, jnp.full((pad,), n, jnp.int32)])
    b1r = b1.reshape(1, -1)
    b2r = b2.reshape(1, -1)
    boutr = bout.reshape(1, -1)

    degp = _sc_degree(n_tab, e_pad)(dst_p).reshape(_NC, n_tab)
    dinv, hs1 = _tc_mm1(degp, x, W1, n_tab)
    acc1 = _sc_aggregate(h, n_tab, e_pad)(hs1, src_p, dst_p)
    hs2 = _tc_layer(acc1.reshape(_NC, n_tab, h), hs1, dinv, b1r, W2)
    acc2 = _sc_aggregate(h, n_tab, e_pad)(hs2, src_p, dst_p)
    q = _tc_head(acc2.reshape(_NC, n_tab, h), hs2, dinv, b2r, Wout, boutr)
    return q
